# baseline (speedup 1.0000x reference)
import functools

import jax
import jax.numpy as jnp
from jax import lax
from jax.experimental import pallas as pl
from jax.experimental.pallas import tpu as pltpu
from jax.experimental.pallas import tpu_sc as plsc

B = 16384
NUM_FIELDS = 26
FIELD_VOCAB = 100000
D = 16
HID = 256
GROUPS = NUM_FIELDS + 2

NC = 2
NS = 16
NW = NC * NS
BW = B // NW

BS = 512


def _sc_gather(user_idx, item_idx, feat_idx, user_tbl, item_tbl, feat_flat):
  mesh = plsc.VectorSubcoreMesh(core_axis_name="c", subcore_axis_name="s")

  @functools.partial(
      pl.kernel,
      mesh=mesh,
      out_type=jax.ShapeDtypeStruct((GROUPS, B, D), jnp.float32),
      scratch_types=[
          pltpu.VMEM((BW,), jnp.int32),
          pltpu.VMEM((BW, D), jnp.float32),
          pltpu.SemaphoreType.DMA,
      ],
      compiler_params=pltpu.CompilerParams(use_tc_tiling_on_sc=False),
  )
  def k(user_h, item_h, feat_h, ut_h, it_h, ft_h, out_h, idx_v, rows_v, sem):
    wid = lax.axis_index("s") * NC + lax.axis_index("c")
    base = wid * BW
    pltpu.sync_copy(user_h.at[pl.ds(base, BW)], idx_v)
    pltpu.async_copy(ut_h.at[idx_v], rows_v, sem).wait()
    pltpu.sync_copy(rows_v, out_h.at[0, pl.ds(base, BW)])
    pltpu.sync_copy(item_h.at[pl.ds(base, BW)], idx_v)
    pltpu.async_copy(it_h.at[idx_v], rows_v, sem).wait()
    pltpu.sync_copy(rows_v, out_h.at[1, pl.ds(base, BW)])

    def body(f, carry):
      pltpu.sync_copy(feat_h.at[f, pl.ds(base, BW)], idx_v)
      pltpu.async_copy(ft_h.at[idx_v], rows_v, sem).wait()
      pltpu.sync_copy(rows_v, out_h.at[f + 2, pl.ds(base, BW)])
      return carry

    lax.fori_loop(0, NUM_FIELDS, body, 0)

  return k(user_idx, item_idx, feat_idx, user_tbl, item_tbl, feat_flat)


def _tc_mlp(emb, w1t, b1r, w2t, b2r):
  def body(emb_ref, w1t_ref, b1_ref, w2t_ref, b2_ref, out_ref):
    acc = jnp.broadcast_to(b1_ref[...], (BS, HID))
    fm = jnp.zeros((BS, 1), jnp.float32)
    for g in range(GROUPS):
      xg = emb_ref[g]
      w1g = w1t_ref[pl.ds(g * D, D), :]
      acc = acc + jax.lax.dot_general(
          xg, w1g, (((1,), (0,)), ((), ())),
          preferred_element_type=jnp.float32)
      fm = fm + jnp.sum(xg, axis=1, keepdims=True)
    h = jnp.maximum(acc, 0.0)
    deep = jax.lax.dot_general(
        h, w2t_ref[...], (((1,), (0,)), ((), ())),
        preferred_element_type=jnp.float32) + b2_ref[...]
    out_ref[...] = jax.nn.sigmoid(fm + deep) * 10.0

  return pl.pallas_call(
      body,
      grid=(B // BS,),
      in_specs=[
          pl.BlockSpec((GROUPS, BS, D), lambda i: (0, i, 0)),
          pl.BlockSpec((GROUPS * D, HID), lambda i: (0, 0)),
          pl.BlockSpec((1, HID), lambda i: (0, 0)),
          pl.BlockSpec((HID, 1), lambda i: (0, 0)),
          pl.BlockSpec((1, 1), lambda i: (0, 0)),
      ],
      out_specs=pl.BlockSpec((BS, 1), lambda i: (i, 0)),
      out_shape=jax.ShapeDtypeStruct((B, 1), jnp.float32),
  )(emb, w1t, b1r, w2t, b2r)


def kernel(user, item, feature, user_table, item_table, feat_tables, W1, b1, W2, b2):
  user = user.astype(jnp.int32)
  item = item.astype(jnp.int32)
  feat_idx = (feature.astype(jnp.int32).T
              + (jnp.arange(NUM_FIELDS, dtype=jnp.int32) * FIELD_VOCAB)[:, None])
  feat_flat = feat_tables.reshape(NUM_FIELDS * FIELD_VOCAB, D)

  emb = _sc_gather(user, item, feat_idx, user_table, item_table, feat_flat)

  w1t = W1.T
  b1r = b1.reshape(1, HID)
  w2t = W2.T
  b2r = b2.reshape(1, 1)
  return _tc_mlp(emb, w1t, b1r, w2t, b2r)


# speedup vs baseline: 1.0119x; 1.0119x over previous
import functools

import jax
import jax.numpy as jnp
from jax import lax
from jax.experimental import pallas as pl
from jax.experimental.pallas import tpu as pltpu
from jax.experimental.pallas import tpu_sc as plsc

B = 16384
NUM_FIELDS = 26
FIELD_VOCAB = 100000
D = 16
HID = 256
GROUPS = NUM_FIELDS + 2

NC = 2
NS = 16
NW = NC * NS
BW = B // NW

BS = 512


NCHUNK = 13
C = (NUM_FIELDS * B) // (NW * NCHUNK)
FW = NCHUNK * C
K = 3


def _sc_gather(user_idx, item_idx, feat_idx, user_tbl, item_tbl, feat_flat):
  mesh = plsc.VectorSubcoreMesh(core_axis_name="c", subcore_axis_name="s")

  @functools.partial(
      pl.kernel,
      mesh=mesh,
      out_type=jax.ShapeDtypeStruct((GROUPS * B, D), jnp.float32),
      scratch_types=[
          pltpu.VMEM((BW,), jnp.int32),
          pltpu.VMEM((BW,), jnp.int32),
          pltpu.VMEM((FW,), jnp.int32),
          pltpu.VMEM((BW, D), jnp.float32),
          pltpu.VMEM((BW, D), jnp.float32),
          pltpu.VMEM((K, C, D), jnp.float32),
          pltpu.SemaphoreType.DMA,
          pltpu.SemaphoreType.DMA,
          pltpu.SemaphoreType.DMA,
          pltpu.SemaphoreType.DMA,
          pltpu.SemaphoreType.DMA((K,)),
          pltpu.SemaphoreType.DMA((K,)),
      ],
      compiler_params=pltpu.CompilerParams(use_tc_tiling_on_sc=False),
  )
  def k(user_h, item_h, feat_h, ut_h, it_h, ft_h, out_h,
        uidx, iidx, fidx, ubuf, ibuf, ring,
        usem, isem, uosem, iosem, gsem, osem):
    wid = lax.axis_index("s") * NC + lax.axis_index("c")
    base = wid * BW
    fbase = wid * FW
    pltpu.sync_copy(user_h.at[pl.ds(base, BW)], uidx)
    pltpu.sync_copy(item_h.at[pl.ds(base, BW)], iidx)
    ug = pltpu.async_copy(ut_h.at[uidx], ubuf, usem)
    ig = pltpu.async_copy(it_h.at[iidx], ibuf, isem)
    pltpu.sync_copy(feat_h.at[pl.ds(fbase, FW)], fidx)
    g = [None] * NCHUNK
    o = [None] * NCHUNK
    for c in range(K):
      g[c] = pltpu.async_copy(
          ft_h.at[fidx.at[pl.ds(c * C, C)]], ring.at[c], gsem.at[c])
    ug.wait()
    uo = pltpu.async_copy(ubuf, out_h.at[pl.ds(base, BW)], uosem)
    ig.wait()
    io = pltpu.async_copy(ibuf, out_h.at[pl.ds(B + base, BW)], iosem)
    for c in range(NCHUNK):
      b = c % K
      g[c].wait()
      o[c] = pltpu.async_copy(
          ring.at[b], out_h.at[pl.ds(2 * B + fbase + c * C, C)], osem.at[b])
      nc = c + K
      if nc < NCHUNK:
        o[c].wait()
        g[nc] = pltpu.async_copy(
            ft_h.at[fidx.at[pl.ds(nc * C, C)]], ring.at[b], gsem.at[b])
    for c in range(NCHUNK - K, NCHUNK):
      o[c].wait()
    uo.wait()
    io.wait()

  return k(user_idx, item_idx, feat_idx, user_tbl, item_tbl, feat_flat)


def _tc_mlp(emb, w1t, b1r, w2t, b2r):
  def body(emb_ref, w1t_ref, b1_ref, w2t_ref, b2_ref, out_ref):
    acc = jnp.broadcast_to(b1_ref[...], (BS, HID))
    fm = jnp.zeros((BS, 1), jnp.float32)
    for g in range(GROUPS):
      xg = emb_ref[g]
      w1g = w1t_ref[pl.ds(g * D, D), :]
      acc = acc + jax.lax.dot_general(
          xg, w1g, (((1,), (0,)), ((), ())),
          preferred_element_type=jnp.float32)
      fm = fm + jnp.sum(xg, axis=1, keepdims=True)
    h = jnp.maximum(acc, 0.0)
    deep = jax.lax.dot_general(
        h, w2t_ref[...], (((1,), (0,)), ((), ())),
        preferred_element_type=jnp.float32) + b2_ref[...]
    out_ref[...] = jax.nn.sigmoid(fm + deep) * 10.0

  return pl.pallas_call(
      body,
      grid=(B // BS,),
      in_specs=[
          pl.BlockSpec((GROUPS, BS, D), lambda i: (0, i, 0)),
          pl.BlockSpec((GROUPS * D, HID), lambda i: (0, 0)),
          pl.BlockSpec((1, HID), lambda i: (0, 0)),
          pl.BlockSpec((HID, 1), lambda i: (0, 0)),
          pl.BlockSpec((1, 1), lambda i: (0, 0)),
      ],
      out_specs=pl.BlockSpec((BS, 1), lambda i: (i, 0)),
      out_shape=jax.ShapeDtypeStruct((B, 1), jnp.float32),
  )(emb, w1t, b1r, w2t, b2r)


def kernel(user, item, feature, user_table, item_table, feat_tables, W1, b1, W2, b2):
  user = user.astype(jnp.int32)
  item = item.astype(jnp.int32)
  feat_idx = (feature.astype(jnp.int32).T
              + (jnp.arange(NUM_FIELDS, dtype=jnp.int32) * FIELD_VOCAB)[:, None]
              ).reshape(NUM_FIELDS * B)
  feat_flat = feat_tables.reshape(NUM_FIELDS * FIELD_VOCAB, D)

  emb = _sc_gather(user, item, feat_idx, user_table, item_table,
                   feat_flat).reshape(GROUPS, B, D)

  w1t = W1.T
  b1r = b1.reshape(1, HID)
  w2t = W2.T
  b2r = b2.reshape(1, 1)
  return _tc_mlp(emb, w1t, b1r, w2t, b2r)


# speedup vs baseline: 2.3888x; 2.3608x over previous
import functools

import jax
import jax.numpy as jnp
from jax import lax
from jax.experimental import pallas as pl
from jax.experimental.pallas import tpu as pltpu
from jax.experimental.pallas import tpu_sc as plsc

B = 16384
NUM_FIELDS = 26
NUM_USERS = 1000000
NUM_ITEMS = 1000000
FIELD_VOCAB = 100000
D = 16
HID = 256
GROUPS = NUM_FIELDS + 2

NC = 2
NS = 16
NW = NC * NS
BW = B // NW

NCHUNK = 13
C = (NUM_FIELDS * B) // (NW * NCHUNK)
FW = NCHUNK * C
K = 3

BS = 512


def _sc_gather(user_idx, item_idx, feat_idx, user_tbl, item_tbl, feat_flat):
  mesh = plsc.VectorSubcoreMesh(core_axis_name="c", subcore_axis_name="s")

  @functools.partial(
      pl.kernel,
      mesh=mesh,
      out_type=jax.ShapeDtypeStruct((GROUPS * B, D), jnp.float32),
      scratch_types=[
          pltpu.VMEM((BW,), jnp.int32),
          pltpu.VMEM((BW,), jnp.int32),
          pltpu.VMEM((FW,), jnp.int32),
          pltpu.VMEM((BW, D), jnp.float32),
          pltpu.VMEM((BW, D), jnp.float32),
          pltpu.VMEM((K, C, D), jnp.float32),
          pltpu.SemaphoreType.DMA,
          pltpu.SemaphoreType.DMA,
          pltpu.SemaphoreType.DMA,
          pltpu.SemaphoreType.DMA,
          pltpu.SemaphoreType.DMA((K,)),
          pltpu.SemaphoreType.DMA((K,)),
      ],
      compiler_params=pltpu.CompilerParams(use_tc_tiling_on_sc=False),
  )
  def k(user_h, item_h, feat_h, ut_h, it_h, ft_h, out_h,
        uidx, iidx, fidx, ubuf, ibuf, ring,
        usem, isem, uosem, iosem, gsem, osem):
    wid = lax.axis_index("s") * NC + lax.axis_index("c")
    base = wid * BW
    fbase = wid * FW
    pltpu.sync_copy(user_h.at[pl.ds(base, BW)], uidx)
    pltpu.sync_copy(item_h.at[pl.ds(base, BW)], iidx)
    ug = pltpu.async_copy(ut_h.at[uidx], ubuf, usem)
    ig = pltpu.async_copy(it_h.at[iidx], ibuf, isem)
    pltpu.sync_copy(feat_h.at[pl.ds(fbase, FW)], fidx)
    g = [None] * NCHUNK
    o = [None] * NCHUNK
    for c in range(K):
      g[c] = pltpu.async_copy(
          ft_h.at[fidx.at[pl.ds(c * C, C)]], ring.at[c], gsem.at[c])
    ug.wait()
    uo = pltpu.async_copy(ubuf, out_h.at[pl.ds(base, BW)], uosem)
    ig.wait()
    io = pltpu.async_copy(ibuf, out_h.at[pl.ds(B + base, BW)], iosem)
    for c in range(NCHUNK):
      b = c % K
      g[c].wait()
      o[c] = pltpu.async_copy(
          ring.at[b], out_h.at[pl.ds(2 * B + fbase + c * C, C)], osem.at[b])
      nc = c + K
      if nc < NCHUNK:
        o[c].wait()
        g[nc] = pltpu.async_copy(
            ft_h.at[fidx.at[pl.ds(nc * C, C)]], ring.at[b], gsem.at[b])
    for c in range(NCHUNK - K, NCHUNK):
      o[c].wait()
    uo.wait()
    io.wait()

  return k(user_idx, item_idx, feat_idx, user_tbl, item_tbl, feat_flat)


def _tc_mlp(emb, w1t, b1r, w2t, b2r):
  def body(emb_ref, w1t_ref, b1_ref, w2t_ref, b2_ref, out_ref):
    acc = jnp.broadcast_to(b1_ref[...], (BS, HID))
    fm = jnp.zeros((BS, 1), jnp.float32)
    for g in range(GROUPS):
      xg = emb_ref[g]
      w1g = w1t_ref[pl.ds(g * D, D), :]
      acc = acc + jax.lax.dot_general(
          xg, w1g, (((1,), (0,)), ((), ())),
          preferred_element_type=jnp.float32)
      fm = fm + jnp.sum(xg, axis=1, keepdims=True)
    h = jnp.maximum(acc, 0.0)
    deep = jax.lax.dot_general(
        h, w2t_ref[...], (((1,), (0,)), ((), ())),
        preferred_element_type=jnp.float32) + b2_ref[...]
    out_ref[...] = jax.nn.sigmoid(fm + deep) * 10.0

  return pl.pallas_call(
      body,
      grid=(B // BS,),
      in_specs=[
          pl.BlockSpec((GROUPS, BS, D), lambda i: (0, i, 0)),
          pl.BlockSpec((GROUPS * D, HID), lambda i: (0, 0)),
          pl.BlockSpec((1, HID), lambda i: (0, 0)),
          pl.BlockSpec((HID, 1), lambda i: (0, 0)),
          pl.BlockSpec((1, 1), lambda i: (0, 0)),
      ],
      out_specs=pl.BlockSpec((BS, 1), lambda i: (i, 0)),
      out_shape=jax.ShapeDtypeStruct((B, 1), jnp.float32),
  )(emb, w1t, b1r, w2t, b2r)


SLABS = 128 // D
CC = 12800


def _lane_perm():
  m = jnp.arange(128, dtype=jnp.int32)
  dst = (m % SLABS) * D + m // SLABS
  return (jnp.arange(128, dtype=jnp.int32)[None, :] == dst[:, None]
          ).astype(jnp.bfloat16)


def _pack_transpose(t3, nchunks):
  G, _, _, S = t3.shape

  def body(in_ref, perm_ref, out_ref):
    x = in_ref[0].reshape(128, CC)
    hi = x.astype(jnp.bfloat16)
    lo = (x - hi.astype(jnp.float32)).astype(jnp.bfloat16)
    p = perm_ref[...]
    dims = (((0,), (0,)), ((), ()))
    out_ref[0] = (
        jax.lax.dot_general(hi, p, dims, preferred_element_type=jnp.float32)
        + jax.lax.dot_general(lo, p, dims, preferred_element_type=jnp.float32))

  return pl.pallas_call(
      body,
      grid=(G, nchunks),
      in_specs=[
          pl.BlockSpec((1, D, SLABS, CC), lambda g, c: (g, 0, 0, c)),
          pl.BlockSpec((128, 128), lambda g, c: (0, 0)),
      ],
      out_specs=pl.BlockSpec((1, CC, 128), lambda g, c: (g * nchunks + c, 0, 0)),
      out_shape=jax.ShapeDtypeStruct((G * nchunks, CC, 128), jnp.float32),
  )(t3, _lane_perm())


def _row_major(table, rows):
  S = rows // SLABS
  nchunks = (S + CC - 1) // CC
  t3 = table.T.reshape(1, D, SLABS, S)
  out = _pack_transpose(t3, nchunks)
  return out.reshape(nchunks * CC * SLABS, D)


def _row_major_fields(ft_t):
  S = FIELD_VOCAB // SLABS
  nchunks = (S + CC - 1) // CC
  t4 = ft_t.reshape(NUM_FIELDS, D, SLABS, S)
  out = _pack_transpose(t4, nchunks)
  return out.reshape(NUM_FIELDS * nchunks * CC * SLABS, D)


def _packed_row(v, S):
  return (v % S) * SLABS + v // S


def kernel(user, item, feature, user_table, item_table, feat_tables, W1, b1, W2, b2):
  su = NUM_USERS // SLABS
  user = _packed_row(user.astype(jnp.int32), su)
  item = _packed_row(item.astype(jnp.int32), NUM_ITEMS // SLABS)
  sf = FIELD_VOCAB // SLABS
  field_stride = ((sf + CC - 1) // CC) * CC * SLABS
  feat_idx = (_packed_row(feature.astype(jnp.int32).T, sf)
              + (jnp.arange(NUM_FIELDS, dtype=jnp.int32) * field_stride)[:, None]
              ).reshape(NUM_FIELDS * B)

  ut_rm = _row_major(user_table, NUM_USERS)
  it_rm = _row_major(item_table, NUM_ITEMS)
  ft_t = jnp.transpose(feat_tables, (0, 2, 1))
  ft_rm = _row_major_fields(ft_t)

  emb = _sc_gather(user, item, feat_idx, ut_rm, it_rm,
                   ft_rm).reshape(GROUPS, B, D)

  w1t = W1.T
  b1r = b1.reshape(1, HID)
  w2t = W2.T
  b2r = b2.reshape(1, 1)
  return _tc_mlp(emb, w1t, b1r, w2t, b2r)
